# baseline (speedup 1.0000x reference)
import functools

import jax
import jax.numpy as jnp
from jax import lax
from jax.experimental import pallas as pl
from jax.experimental.pallas import tpu as pltpu
from jax.experimental.pallas import tpu_sc as plsc

_L = 16
_BB = 8


@functools.lru_cache(maxsize=None)
def _make_sc_sel(C):
    n_chunks = C // _L
    mesh = plsc.VectorSubcoreMesh(core_axis_name="c", subcore_axis_name="s")

    @functools.partial(
        pl.kernel,
        mesh=mesh,
        compiler_params=pltpu.CompilerParams(needs_layout_passes=False),
        out_type=jax.ShapeDtypeStruct((C,), jnp.int32),
        scratch_types=[
            pltpu.VMEM((C,), jnp.float32),
            pltpu.VMEM((C,), jnp.int32),
        ],
    )
    def k(indexes_hbm, sel_hbm, idxs_v, sel_v):
        wid = lax.axis_index("s") * 2 + lax.axis_index("c")
        pltpu.sync_copy(indexes_hbm, idxs_v)

        iota = lax.iota(jnp.int32, _L)
        one = jnp.int32(1)
        zero = jnp.int32(0)

        acc = jnp.zeros((_L,), jnp.int32)
        for c in range(n_chunks):
            v = idxs_v[pl.ds(c * _L, _L)]
            acc = acc + jnp.where(v != 0.0, one, zero)
        total_nz = zero
        for j in range(_L):
            total_nz = total_nz + acc[j]

        nz = zero
        z = zero
        for c in range(n_chunks):
            v = idxs_v[pl.ds(c * _L, _L)]
            posvec = jnp.zeros((_L,), jnp.int32)
            for j in range(_L):
                mj = v[j] != 0.0
                pos_j = jnp.where(mj, nz, total_nz + z)
                posvec = jnp.where(iota == j, pos_j, posvec)
                nz = nz + jnp.where(mj, one, zero)
                z = z + jnp.where(mj, zero, one)
            plsc.store_scatter(sel_v, [posvec], iota + (c * _L))

        @pl.when(wid == 0)
        def _():
            pltpu.sync_copy(sel_v, sel_hbm)

    return k


@functools.lru_cache(maxsize=None)
def _make_tc_gather(B, C, H, W):
    assert B % _BB == 0

    def body(sel_smem, in_ref, out_ref):
        out_ref[...] = in_ref[...]

    grid_spec = pltpu.PrefetchScalarGridSpec(
        num_scalar_prefetch=1,
        grid=(B // _BB, C),
        in_specs=[
            pl.BlockSpec(
                (_BB, 1, H, W), lambda bb, c, sel_ref: (bb, sel_ref[c], 0, 0)
            ),
        ],
        out_specs=pl.BlockSpec(
            (_BB, 1, H, W), lambda bb, c, sel_ref: (bb, c, 0, 0)
        ),
    )
    return pl.pallas_call(
        body,
        grid_spec=grid_spec,
        out_shape=jax.ShapeDtypeStruct((B, C, H, W), jnp.float32),
    )


def kernel(input_tensor, indexes):
    B, C, H, W = input_tensor.shape
    sel = _make_sc_sel(C)(indexes)
    return _make_tc_gather(B, C, H, W)(sel, input_tensor)


# speedup vs baseline: 1.8574x; 1.8574x over previous
import functools

import jax
import jax.numpy as jnp
from jax import lax
from jax.experimental import pallas as pl
from jax.experimental.pallas import tpu as pltpu
from jax.experimental.pallas import tpu_sc as plsc

_L = 16
_BB = 8


@functools.lru_cache(maxsize=None)
def _make_sc_sel(C):
    n_chunks = C // _L
    mesh = plsc.VectorSubcoreMesh(core_axis_name="c", subcore_axis_name="s")

    @functools.partial(
        pl.kernel,
        mesh=mesh,
        compiler_params=pltpu.CompilerParams(needs_layout_passes=False),
        out_type=jax.ShapeDtypeStruct((C,), jnp.int32),
        scratch_types=[
            pltpu.VMEM((C,), jnp.float32),
            pltpu.VMEM((C,), jnp.int32),
        ],
    )
    def k(indexes_hbm, sel_hbm, idxs_v, sel_v):
        wid = lax.axis_index("s") * 2 + lax.axis_index("c")
        pltpu.sync_copy(indexes_hbm, idxs_v)

        iota = lax.iota(jnp.int32, _L)
        one = jnp.int32(1)
        zero = jnp.int32(0)

        acc = jnp.zeros((_L,), jnp.int32)
        for c in range(n_chunks):
            v = idxs_v[pl.ds(c * _L, _L)]
            acc = acc + jnp.where(v != 0.0, one, zero)
        total_nz = zero
        for j in range(_L):
            total_nz = total_nz + acc[j]

        nz = zero
        z = zero
        for c in range(n_chunks):
            v = idxs_v[pl.ds(c * _L, _L)]
            posvec = jnp.zeros((_L,), jnp.int32)
            for j in range(_L):
                mj = v[j] != 0.0
                pos_j = jnp.where(mj, nz, total_nz + z)
                posvec = jnp.where(iota == j, pos_j, posvec)
                nz = nz + jnp.where(mj, one, zero)
                z = z + jnp.where(mj, zero, one)
            plsc.store_scatter(sel_v, [posvec], iota + (c * _L))

        @pl.when(wid == 0)
        def _():
            pltpu.sync_copy(sel_v, sel_hbm)

    return k


@functools.lru_cache(maxsize=None)
def _make_tc_gather(B, C, H, W):
    def body(sel_smem, in_ref, out_ref):
        def cp(c, _):
            out_ref[:, pl.ds(c, 1)] = in_ref[:, pl.ds(sel_smem[c], 1)]
            return _

        lax.fori_loop(0, C, cp, 0)

    grid_spec = pltpu.PrefetchScalarGridSpec(
        num_scalar_prefetch=1,
        grid=(B,),
        in_specs=[
            pl.BlockSpec((1, C, H, W), lambda b, sel_ref: (b, 0, 0, 0)),
        ],
        out_specs=pl.BlockSpec((1, C, H, W), lambda b, sel_ref: (b, 0, 0, 0)),
    )
    return pl.pallas_call(
        body,
        grid_spec=grid_spec,
        out_shape=jax.ShapeDtypeStruct((B, C, H, W), jnp.float32),
    )


def kernel(input_tensor, indexes):
    B, C, H, W = input_tensor.shape
    sel = _make_sc_sel(C)(indexes)
    return _make_tc_gather(B, C, H, W)(sel, input_tensor)
